# Initial kernel scaffold; baseline (speedup 1.0000x reference)
import jax
import jax.numpy as jnp
from jax.experimental import pallas as pl


def kernel(atoms, pos, edge_index, edge_attr, node_attr, batch, additional_message_features, params):
    raise NotImplementedError("write your pallas kernel here")


# speedup vs baseline: 2.4405x; 2.4405x over previous
import functools

import jax
import jax.numpy as jnp
import numpy as np
from jax import lax
from jax.experimental import pallas as pl
from jax.experimental.pallas import tpu as pltpu
from jax.experimental.pallas import tpu_sc as plsc

N = 10000
E = 160000
H = 64
A = 4
P = 64
AH = A * H
W = 2 * H

NC = 2
NS = 16
NW = NC * NS
CH = 128
NCHUNK = E // CH

BE = 2000
BN = 1000

_f32 = jnp.float32


def _np_consts():
    R = np.zeros((A, AH), np.float32)
    S = np.zeros((AH, H), np.float32)
    T = np.zeros((H, AH), np.float32)
    for a in range(A):
        R[a, a * H:(a + 1) * H] = 1.0
        for o in range(H):
            S[a * H + o, o] = 1.0
            T[o, a * H + o] = 1.0
    return R, S, T

_R_np, _S_np, _T_np = _np_consts()


def _silu(x):
    return x * jax.nn.sigmoid(x)


def _dot(a, b):
    return jnp.dot(a, b, preferred_element_type=_f32)


@functools.cache
def _sc_mesh():
    return plsc.VectorSubcoreMesh(core_axis_name="c", subcore_axis_name="s",
                                  num_cores=NC, num_subcores=NS)


def _gather_body(xl_hbm, xr_hbm, src_hbm, dst_hbm, xij_out,
                 idx_d, idx_s, rows, sem0, sem1):
    c = lax.axis_index("c")
    s = lax.axis_index("s")
    wid = s * NC + c
    nj = jnp.where(wid < NCHUNK - (NCHUNK // NW) * NW, NCHUNK // NW + 1,
                   NCHUNK // NW)

    def body(j, carry):
        off = (wid + j * NW) * CH
        c1 = pltpu.async_copy(dst_hbm.at[pl.ds(off, CH)], idx_d, sem0)
        c2 = pltpu.async_copy(src_hbm.at[pl.ds(off, CH)], idx_s, sem1)
        c1.wait()
        c2.wait()
        g1 = pltpu.async_copy(xl_hbm.at[idx_d], rows, sem0)
        g1.wait()
        g2 = pltpu.async_copy(xr_hbm.at[idx_s], rows, sem1, add=True)
        g2.wait()
        w1 = pltpu.async_copy(rows, xij_out.at[pl.ds(off, CH)], sem0)
        w1.wait()
        return carry

    lax.fori_loop(0, nj, body, 0)


def _gather_call(xl, xr, src, dst):
    fn = pl.kernel(
        _gather_body,
        out_type=jax.ShapeDtypeStruct((E, W), _f32),
        mesh=_sc_mesh(),
        scratch_types=[
            pltpu.VMEM((CH,), jnp.int32),
            pltpu.VMEM((CH,), jnp.int32),
            pltpu.VMEM((CH, W), _f32),
            pltpu.SemaphoreType.DMA,
            pltpu.SemaphoreType.DMA,
        ],
    )
    return fn(xl, xr, src, dst)


ESC = E // NC
CSC = ESC // CH
RT_A = 632
RT_L = N - RT_A * (NS - 1)


def _scatter_body(m2_hbm, dst_hbm, zeros_hbm, out_hbm,
                  idx_v, rows_v, sem0, acc):
    c = lax.axis_index("c")
    s = lax.axis_index("s")
    row0 = pl.multiple_of(s * RT_A, 8)

    @pl.when(s < NS - 1)
    def _():
        pltpu.sync_copy(zeros_hbm.at[pl.ds(row0, RT_A)],
                        acc.at[pl.ds(row0, RT_A)])

    @pl.when(s == NS - 1)
    def _():
        pltpu.sync_copy(zeros_hbm.at[pl.ds((NS - 1) * RT_A, RT_L)],
                        acc.at[pl.ds((NS - 1) * RT_A, RT_L)])

    plsc.subcore_barrier()
    base = c * CSC
    nj = jnp.where(s < CSC - (CSC // NS) * NS, CSC // NS + 1, CSC // NS)

    def body(j, carry):
        off = pl.multiple_of((base + s + j * NS) * CH, CH)
        pltpu.sync_copy(dst_hbm.at[pl.ds(off, CH)], idx_v.at[0])
        pltpu.sync_copy(m2_hbm.at[pl.ds(off, CH)], rows_v)
        pltpu.sync_copy(rows_v, acc.at[idx_v.at[0]], add=True)
        return carry

    lax.fori_loop(0, nj, body, 0)
    plsc.subcore_barrier()

    @pl.when(s < NS - 1)
    def _():
        pltpu.sync_copy(acc.at[pl.ds(row0, RT_A)],
                        out_hbm.at[c, pl.ds(row0, RT_A)])

    @pl.when(s == NS - 1)
    def _():
        pltpu.sync_copy(acc.at[pl.ds((NS - 1) * RT_A, RT_L)],
                        out_hbm.at[c, pl.ds((NS - 1) * RT_A, RT_L)])


def _scatter_call(m2, dst, zeros_nh):
    fn = pl.kernel(
        _scatter_body,
        out_type=jax.ShapeDtypeStruct((NC, N, W), _f32),
        mesh=_sc_mesh(),
        scratch_types=[
            pltpu.VMEM((1, CH), jnp.int32),
            pltpu.VMEM((CH, W), _f32),
            pltpu.SemaphoreType.DMA,
            pltpu.VMEM_SHARED((N, W), _f32),
        ],
    )
    return fn(m2, dst, zeros_nh)


def _embed_body(at_ref, na_ref, wemb_ref, bemb_ref, outl_ref, outr_ref):
    v = at_ref[...] * _dot(na_ref[...], wemb_ref[...]) + bemb_ref[...]
    z = jnp.zeros_like(v)
    outl_ref[...] = jnp.concatenate([v, z], axis=1)
    outr_ref[...] = jnp.concatenate([z, v], axis=1)


def _edge_body(xij_ref, ea_ref, amf_ref, wds_ref, w3_ref, r_ref,
               s_ref, t_ref, wm2_ref, bm1_ref, bm2_ref, out_ref):
    Y = _dot(xij_ref[...], wds_ref[...]) + _dot(amf_ref[...], w3_ref[...])
    EA = _dot(ea_ref[...], r_ref[...])
    m1 = _silu(_dot(Y * EA, s_ref[...]) + bm1_ref[...])
    m2 = _silu(_dot(_dot(m1, t_ref[...]) * EA, wm2_ref[...]) + bm2_ref[...])
    out_ref[...] = jnp.concatenate([m2, jnp.zeros_like(m2)], axis=1)


def _node_body(xl_ref, agg_ref, na_ref, wu1_ref, wu2_ref, r_ref, s_ref,
               t_ref, bu1_ref, bu2_ref, outl_ref, outr_ref):
    x = xl_ref[:, :H]
    agg = (agg_ref[0] + agg_ref[1])[:, :H]
    u_in = jnp.concatenate([x, agg], axis=1)
    NAm = _dot(na_ref[...], r_ref[...])
    t = _silu(_dot(_dot(u_in, wu1_ref[...]) * NAm, s_ref[...]) + bu1_ref[...])
    u = _dot(_dot(t, t_ref[...]) * NAm, wu2_ref[...]) + bu2_ref[...]
    xn = x + u
    z = jnp.zeros_like(xn)
    outl_ref[...] = jnp.concatenate([xn, z], axis=1)
    outr_ref[...] = jnp.concatenate([z, xn], axis=1)


def _readout_body(xl_ref, na_ref, wp1_ref, wp2_ref, r_ref, s_ref, bp1_ref,
                  bp2_ref, wq1_ref, bq1_ref, wq2_ref, bq2_ref, out_ref, acc):
    i = pl.program_id(0)

    @pl.when(i == 0)
    def _():
        acc[...] = jnp.zeros_like(acc)

    NAm = _dot(na_ref[...], r_ref[...])
    t1 = _silu(_dot(_dot(xl_ref[:, :H], wp1_ref[...]) * NAm, s_ref[...])
               + bp1_ref[...])
    t2 = _dot(_dot(t1, wp2_ref[...]) * NAm, s_ref[...]) + bp2_ref[...]
    acc[...] += jnp.sum(t2, axis=0, keepdims=True)

    @pl.when(i == pl.num_programs(0) - 1)
    def _():
        pooled = acc[...] / float(N)
        h = _silu(_dot(pooled, wq1_ref[...]) + bq1_ref[...])
        out_ref[...] = _dot(h, wq2_ref[...]) + bq2_ref[...]


def _full(shape):
    nd = len(shape)
    return pl.BlockSpec(shape, lambda i: (0,) * nd)


def _embed_call(at2, na, wemb, bemb):
    g = N // BN
    return pl.pallas_call(
        _embed_body,
        grid=(g,),
        in_specs=[pl.BlockSpec((BN, 1), lambda i: (i, 0)),
                  pl.BlockSpec((BN, A), lambda i: (i, 0)),
                  _full((A, H)), _full((1, H))],
        out_specs=[pl.BlockSpec((BN, W), lambda i: (i, 0)),
                   pl.BlockSpec((BN, W), lambda i: (i, 0))],
        out_shape=[jax.ShapeDtypeStruct((N, W), _f32),
                   jax.ShapeDtypeStruct((N, W), _f32)],
    )(at2, na, wemb, bemb)


def _edge_call(xij, ea, amf, wds, w3, r, s, t, wm2, bm1, bm2):
    g = E // BE
    return pl.pallas_call(
        _edge_body,
        grid=(g,),
        in_specs=[pl.BlockSpec((BE, W), lambda i: (i, 0)),
                  pl.BlockSpec((BE, A), lambda i: (i, 0)),
                  pl.BlockSpec((BE, 1), lambda i: (i, 0)),
                  _full((W, AH)), _full((1, AH)), _full((A, AH)),
                  _full((AH, H)), _full((H, AH)), _full((AH, H)),
                  _full((1, H)), _full((1, H))],
        out_specs=pl.BlockSpec((BE, W), lambda i: (i, 0)),
        out_shape=jax.ShapeDtypeStruct((E, W), _f32),
    )(xij, ea, amf, wds, w3, r, s, t, wm2, bm1, bm2)


def _node_call(xl, aggp, na, wu1, wu2, r, s, t, bu1, bu2):
    g = N // BN
    return pl.pallas_call(
        _node_body,
        grid=(g,),
        in_specs=[pl.BlockSpec((BN, W), lambda i: (i, 0)),
                  pl.BlockSpec((NC, BN, W), lambda i: (0, i, 0)),
                  pl.BlockSpec((BN, A), lambda i: (i, 0)),
                  _full((2 * H, AH)), _full((AH, H)), _full((A, AH)),
                  _full((AH, H)), _full((H, AH)),
                  _full((1, H)), _full((1, H))],
        out_specs=[pl.BlockSpec((BN, W), lambda i: (i, 0)),
                   pl.BlockSpec((BN, W), lambda i: (i, 0))],
        out_shape=[jax.ShapeDtypeStruct((N, W), _f32),
                   jax.ShapeDtypeStruct((N, W), _f32)],
    )(xl, aggp, na, wu1, wu2, r, s, t, bu1, bu2)


def _readout_call(xl, na, wp1, wp2, r, s, bp1, bp2, wq1, bq1, wq2, bq2):
    g = N // BN
    return pl.pallas_call(
        _readout_body,
        grid=(g,),
        in_specs=[pl.BlockSpec((BN, W), lambda i: (i, 0)),
                  pl.BlockSpec((BN, A), lambda i: (i, 0)),
                  _full((H, AH)), _full((H, AH)), _full((A, AH)),
                  _full((AH, H)), _full((1, H)), _full((1, P)),
                  _full((P, P)), _full((1, P)), _full((P, 1)), _full((1, 1))],
        out_specs=_full((1, 1)),
        out_shape=jax.ShapeDtypeStruct((1, 1), _f32),
        scratch_shapes=[pltpu.VMEM((1, P), _f32)],
    )(xl, na, wp1, wp2, r, s, bp1, bp2, wq1, bq1, wq2, bq2)


def kernel(atoms, pos, edge_index, edge_attr, node_attr, batch,
           additional_message_features, params):
    node_attr = node_attr.at[:, 0].set(1.0)
    src = edge_index[0]
    dst = edge_index[1]
    amf = additional_message_features

    R = jnp.asarray(_R_np)
    S = jnp.asarray(_S_np)
    T = jnp.asarray(_T_np)
    zeros_nh = jnp.zeros((N, W), _f32)

    xl, xr = _embed_call(atoms[:, None].astype(_f32), node_attr,
                         params['W_emb'][0], params['b_emb'][None, :])

    for l in range(2):
        p = params['layer%d' % l]
        wds = p['Wm1'][:2 * H].reshape(2 * H, AH)
        w3 = p['Wm1'][2 * H:].reshape(1, AH)
        wm2 = p['Wm2'].transpose(1, 0, 2).reshape(AH, H)
        wu1 = p['Wu1'].reshape(2 * H, AH)
        wu2 = p['Wu2'].transpose(1, 0, 2).reshape(AH, H)

        xij = _gather_call(xl, xr, src, dst)
        m2 = _edge_call(xij, edge_attr, amf, wds, w3, R, S, T, wm2,
                        p['bm1'][None, :], p['bm2'][None, :])
        aggp = _scatter_call(m2, dst, zeros_nh)
        xl, xr = _node_call(xl, aggp, node_attr, wu1, wu2, R, S, T,
                            p['bu1'][None, :], p['bu2'][None, :])

    out = _readout_call(
        xl, node_attr,
        params['W_pre1'].reshape(H, AH), params['W_pre2'].reshape(H, A * P),
        R, S, params['b_pre1'][None, :], params['b_pre2'][None, :],
        params['W_post1'], params['b_post1'][None, :],
        params['W_post2'], params['b_post2'][None, :])
    return out


# speedup vs baseline: 2.9121x; 1.1932x over previous
import functools

import jax
import jax.numpy as jnp
import numpy as np
from jax import lax
from jax.experimental import pallas as pl
from jax.experimental.pallas import tpu as pltpu
from jax.experimental.pallas import tpu_sc as plsc

N = 10000
E = 160000
H = 64
A = 4
P = 64
AH = A * H
W = 2 * H

NC = 2
NS = 16
NW = NC * NS
CH = 128
NCHUNK = E // CH

BE = 2000
BN = 1000

_f32 = jnp.float32


def _np_consts():
    R = np.zeros((A, AH), np.float32)
    S = np.zeros((AH, H), np.float32)
    T = np.zeros((H, AH), np.float32)
    for a in range(A):
        R[a, a * H:(a + 1) * H] = 1.0
        for o in range(H):
            S[a * H + o, o] = 1.0
            T[o, a * H + o] = 1.0
    return R, S, T

_R_np, _S_np, _T_np = _np_consts()


def _silu(x):
    return x * jax.nn.sigmoid(x)


def _dot(a, b):
    return jnp.dot(a, b, preferred_element_type=_f32)


@functools.cache
def _sc_mesh():
    return plsc.VectorSubcoreMesh(core_axis_name="c", subcore_axis_name="s",
                                  num_cores=NC, num_subcores=NS)


KG = 3
NGRP = 13


def _gather_body(xl_hbm, xr_hbm, src_hbm, dst_hbm, xij_out,
                 idx_d, idx_s, rows, sem0, sem1, sem2):
    sems = (sem0, sem1, sem2)
    c = lax.axis_index("c")
    s = lax.axis_index("s")
    wid = s * NC + c

    def off_of(j):
        return pl.multiple_of((wid + j * NW) * CH, CH)

    def grp(g, carry):
        h_idx = []
        for b in range(KG):
            off = off_of(g * KG + b)
            h1 = pltpu.async_copy(dst_hbm.at[pl.ds(off, CH)], idx_d.at[b],
                                  sems[b])
            h2 = pltpu.async_copy(src_hbm.at[pl.ds(off, CH)], idx_s.at[b],
                                  sems[b])
            h_idx.append((h1, h2))
        h_g = []
        for b in range(KG):
            h_idx[b][0].wait()
            h_idx[b][1].wait()
            h_g.append(pltpu.async_copy(xl_hbm.at[idx_d.at[b]], rows.at[b],
                                        sems[b]))
        h_a = []
        for b in range(KG):
            h_g[b].wait()
            h_a.append(pltpu.async_copy(xr_hbm.at[idx_s.at[b]], rows.at[b],
                                        sems[b], add=True))
        h_w = []
        for b in range(KG):
            h_a[b].wait()
            off = off_of(g * KG + b)
            h_w.append(pltpu.async_copy(rows.at[b],
                                        xij_out.at[pl.ds(off, CH)], sems[b]))
        for b in range(KG):
            h_w[b].wait()
        return carry

    lax.fori_loop(0, NGRP, grp, 0)

    @pl.when(wid < NCHUNK - NGRP * KG * NW)
    def _():
        off = off_of(NGRP * KG)
        pltpu.sync_copy(dst_hbm.at[pl.ds(off, CH)], idx_d.at[0])
        pltpu.sync_copy(src_hbm.at[pl.ds(off, CH)], idx_s.at[0])
        g1 = pltpu.async_copy(xl_hbm.at[idx_d.at[0]], rows.at[0], sem0)
        g1.wait()
        g2 = pltpu.async_copy(xr_hbm.at[idx_s.at[0]], rows.at[0], sem0,
                              add=True)
        g2.wait()
        pltpu.sync_copy(rows.at[0], xij_out.at[pl.ds(off, CH)])


def _gather_call(xl, xr, src, dst):
    fn = pl.kernel(
        _gather_body,
        out_type=jax.ShapeDtypeStruct((E, W), _f32),
        mesh=_sc_mesh(),
        scratch_types=[
            pltpu.VMEM((KG, CH), jnp.int32),
            pltpu.VMEM((KG, CH), jnp.int32),
            pltpu.VMEM((KG, CH, W), _f32),
            pltpu.SemaphoreType.DMA,
            pltpu.SemaphoreType.DMA,
            pltpu.SemaphoreType.DMA,
        ],
    )
    return fn(xl, xr, src, dst)


ESC = E // NC
CSC = ESC // CH
RT_A = 632
RT_L = N - RT_A * (NS - 1)


def _scatter_body(m2_hbm, dst_hbm, zeros_hbm, out_hbm,
                  idx_v, rows_v, sem0, sem1, sem2, acc):
    c = lax.axis_index("c")
    s = lax.axis_index("s")
    row0 = pl.multiple_of(s * RT_A, 8)

    @pl.when(s < NS - 1)
    def _():
        pltpu.sync_copy(zeros_hbm.at[pl.ds(row0, RT_A)],
                        acc.at[pl.ds(row0, RT_A)])

    @pl.when(s == NS - 1)
    def _():
        pltpu.sync_copy(zeros_hbm.at[pl.ds((NS - 1) * RT_A, RT_L)],
                        acc.at[pl.ds((NS - 1) * RT_A, RT_L)])

    plsc.subcore_barrier()
    base = c * CSC
    sems = (sem0, sem1, sem2)

    def off_of(j):
        return pl.multiple_of((base + s + j * NS) * CH, CH)

    def grp(g, carry):
        h_ld = []
        for b in range(KG):
            off = off_of(g * KG + b)
            h1 = pltpu.async_copy(dst_hbm.at[pl.ds(off, CH)], idx_v.at[b],
                                  sems[b])
            h2 = pltpu.async_copy(m2_hbm.at[pl.ds(off, CH)], rows_v.at[b],
                                  sems[b])
            h_ld.append((h1, h2))
        h_a = []
        for b in range(KG):
            h_ld[b][0].wait()
            h_ld[b][1].wait()
            h_a.append(pltpu.async_copy(rows_v.at[b], acc.at[idx_v.at[b]],
                                        sems[b], add=True))
        for b in range(KG):
            h_a[b].wait()
        return carry

    lax.fori_loop(0, NGRP, grp, 0)

    @pl.when(s < CSC - NGRP * KG * NS)
    def _():
        off = off_of(NGRP * KG)
        pltpu.sync_copy(dst_hbm.at[pl.ds(off, CH)], idx_v.at[0])
        pltpu.sync_copy(m2_hbm.at[pl.ds(off, CH)], rows_v.at[0])
        pltpu.sync_copy(rows_v.at[0], acc.at[idx_v.at[0]], add=True)

    plsc.subcore_barrier()

    @pl.when(s < NS - 1)
    def _():
        pltpu.sync_copy(acc.at[pl.ds(row0, RT_A)],
                        out_hbm.at[c, pl.ds(row0, RT_A)])

    @pl.when(s == NS - 1)
    def _():
        pltpu.sync_copy(acc.at[pl.ds((NS - 1) * RT_A, RT_L)],
                        out_hbm.at[c, pl.ds((NS - 1) * RT_A, RT_L)])


def _scatter_call(m2, dst, zeros_nh):
    fn = pl.kernel(
        _scatter_body,
        out_type=jax.ShapeDtypeStruct((NC, N, W), _f32),
        mesh=_sc_mesh(),
        scratch_types=[
            pltpu.VMEM((KG, CH), jnp.int32),
            pltpu.VMEM((KG, CH, W), _f32),
            pltpu.SemaphoreType.DMA,
            pltpu.SemaphoreType.DMA,
            pltpu.SemaphoreType.DMA,
            pltpu.VMEM_SHARED((N, W), _f32),
        ],
    )
    return fn(m2, dst, zeros_nh)


def _embed_body(at_ref, na_ref, wemb_ref, bemb_ref, outl_ref, outr_ref):
    v = at_ref[...] * _dot(na_ref[...], wemb_ref[...]) + bemb_ref[...]
    z = jnp.zeros_like(v)
    outl_ref[...] = jnp.concatenate([v, z], axis=1)
    outr_ref[...] = jnp.concatenate([z, v], axis=1)


def _edge_body(xij_ref, ea_ref, amf_ref, wds_ref, w3_ref, r_ref,
               s_ref, t_ref, wm2_ref, bm1_ref, bm2_ref, out_ref):
    Y = _dot(xij_ref[...], wds_ref[...]) + _dot(amf_ref[...], w3_ref[...])
    EA = _dot(ea_ref[...], r_ref[...])
    m1 = _silu(_dot(Y * EA, s_ref[...]) + bm1_ref[...])
    m2 = _silu(_dot(_dot(m1, t_ref[...]) * EA, wm2_ref[...]) + bm2_ref[...])
    out_ref[...] = jnp.concatenate([m2, jnp.zeros_like(m2)], axis=1)


def _node_body(xl_ref, agg_ref, na_ref, wu1_ref, wu2_ref, r_ref, s_ref,
               t_ref, bu1_ref, bu2_ref, outl_ref, outr_ref):
    x = xl_ref[:, :H]
    agg = (agg_ref[0] + agg_ref[1])[:, :H]
    u_in = jnp.concatenate([x, agg], axis=1)
    NAm = _dot(na_ref[...], r_ref[...])
    t = _silu(_dot(_dot(u_in, wu1_ref[...]) * NAm, s_ref[...]) + bu1_ref[...])
    u = _dot(_dot(t, t_ref[...]) * NAm, wu2_ref[...]) + bu2_ref[...]
    xn = x + u
    z = jnp.zeros_like(xn)
    outl_ref[...] = jnp.concatenate([xn, z], axis=1)
    outr_ref[...] = jnp.concatenate([z, xn], axis=1)


def _readout_body(xl_ref, na_ref, wp1_ref, wp2_ref, r_ref, s_ref, bp1_ref,
                  bp2_ref, wq1_ref, bq1_ref, wq2_ref, bq2_ref, out_ref, acc):
    i = pl.program_id(0)

    @pl.when(i == 0)
    def _():
        acc[...] = jnp.zeros_like(acc)

    NAm = _dot(na_ref[...], r_ref[...])
    t1 = _silu(_dot(_dot(xl_ref[:, :H], wp1_ref[...]) * NAm, s_ref[...])
               + bp1_ref[...])
    t2 = _dot(_dot(t1, wp2_ref[...]) * NAm, s_ref[...]) + bp2_ref[...]
    acc[...] += jnp.sum(t2, axis=0, keepdims=True)

    @pl.when(i == pl.num_programs(0) - 1)
    def _():
        pooled = acc[...] / float(N)
        h = _silu(_dot(pooled, wq1_ref[...]) + bq1_ref[...])
        out_ref[...] = _dot(h, wq2_ref[...]) + bq2_ref[...]


def _full(shape):
    nd = len(shape)
    return pl.BlockSpec(shape, lambda i: (0,) * nd)


def _embed_call(at2, na, wemb, bemb):
    g = N // BN
    return pl.pallas_call(
        _embed_body,
        grid=(g,),
        in_specs=[pl.BlockSpec((BN, 1), lambda i: (i, 0)),
                  pl.BlockSpec((BN, A), lambda i: (i, 0)),
                  _full((A, H)), _full((1, H))],
        out_specs=[pl.BlockSpec((BN, W), lambda i: (i, 0)),
                   pl.BlockSpec((BN, W), lambda i: (i, 0))],
        out_shape=[jax.ShapeDtypeStruct((N, W), _f32),
                   jax.ShapeDtypeStruct((N, W), _f32)],
    )(at2, na, wemb, bemb)


def _edge_call(xij, ea, amf, wds, w3, r, s, t, wm2, bm1, bm2):
    g = E // BE
    return pl.pallas_call(
        _edge_body,
        grid=(g,),
        in_specs=[pl.BlockSpec((BE, W), lambda i: (i, 0)),
                  pl.BlockSpec((BE, A), lambda i: (i, 0)),
                  pl.BlockSpec((BE, 1), lambda i: (i, 0)),
                  _full((W, AH)), _full((1, AH)), _full((A, AH)),
                  _full((AH, H)), _full((H, AH)), _full((AH, H)),
                  _full((1, H)), _full((1, H))],
        out_specs=pl.BlockSpec((BE, W), lambda i: (i, 0)),
        out_shape=jax.ShapeDtypeStruct((E, W), _f32),
    )(xij, ea, amf, wds, w3, r, s, t, wm2, bm1, bm2)


def _node_call(xl, aggp, na, wu1, wu2, r, s, t, bu1, bu2):
    g = N // BN
    return pl.pallas_call(
        _node_body,
        grid=(g,),
        in_specs=[pl.BlockSpec((BN, W), lambda i: (i, 0)),
                  pl.BlockSpec((NC, BN, W), lambda i: (0, i, 0)),
                  pl.BlockSpec((BN, A), lambda i: (i, 0)),
                  _full((2 * H, AH)), _full((AH, H)), _full((A, AH)),
                  _full((AH, H)), _full((H, AH)),
                  _full((1, H)), _full((1, H))],
        out_specs=[pl.BlockSpec((BN, W), lambda i: (i, 0)),
                   pl.BlockSpec((BN, W), lambda i: (i, 0))],
        out_shape=[jax.ShapeDtypeStruct((N, W), _f32),
                   jax.ShapeDtypeStruct((N, W), _f32)],
    )(xl, aggp, na, wu1, wu2, r, s, t, bu1, bu2)


def _readout_call(xl, na, wp1, wp2, r, s, bp1, bp2, wq1, bq1, wq2, bq2):
    g = N // BN
    return pl.pallas_call(
        _readout_body,
        grid=(g,),
        in_specs=[pl.BlockSpec((BN, W), lambda i: (i, 0)),
                  pl.BlockSpec((BN, A), lambda i: (i, 0)),
                  _full((H, AH)), _full((H, AH)), _full((A, AH)),
                  _full((AH, H)), _full((1, H)), _full((1, P)),
                  _full((P, P)), _full((1, P)), _full((P, 1)), _full((1, 1))],
        out_specs=_full((1, 1)),
        out_shape=jax.ShapeDtypeStruct((1, 1), _f32),
        scratch_shapes=[pltpu.VMEM((1, P), _f32)],
    )(xl, na, wp1, wp2, r, s, bp1, bp2, wq1, bq1, wq2, bq2)


def kernel(atoms, pos, edge_index, edge_attr, node_attr, batch,
           additional_message_features, params):
    node_attr = node_attr.at[:, 0].set(1.0)
    src = edge_index[0]
    dst = edge_index[1]
    amf = additional_message_features

    R = jnp.asarray(_R_np)
    S = jnp.asarray(_S_np)
    T = jnp.asarray(_T_np)
    zeros_nh = jnp.zeros((N, W), _f32)

    xl, xr = _embed_call(atoms[:, None].astype(_f32), node_attr,
                         params['W_emb'][0], params['b_emb'][None, :])

    for l in range(2):
        p = params['layer%d' % l]
        wds = p['Wm1'][:2 * H].reshape(2 * H, AH)
        w3 = p['Wm1'][2 * H:].reshape(1, AH)
        wm2 = p['Wm2'].transpose(1, 0, 2).reshape(AH, H)
        wu1 = p['Wu1'].reshape(2 * H, AH)
        wu2 = p['Wu2'].transpose(1, 0, 2).reshape(AH, H)

        xij = _gather_call(xl, xr, src, dst)
        m2 = _edge_call(xij, edge_attr, amf, wds, w3, R, S, T, wm2,
                        p['bm1'][None, :], p['bm2'][None, :])
        aggp = _scatter_call(m2, dst, zeros_nh)
        xl, xr = _node_call(xl, aggp, node_attr, wu1, wu2, R, S, T,
                            p['bu1'][None, :], p['bu2'][None, :])

    out = _readout_call(
        xl, node_attr,
        params['W_pre1'].reshape(H, AH), params['W_pre2'].reshape(H, A * P),
        R, S, params['b_pre1'][None, :], params['b_pre2'][None, :],
        params['W_post1'], params['b_post1'][None, :],
        params['W_post2'], params['b_post2'][None, :])
    return out
